# baseline (speedup 1.0000x reference)
import functools

import jax
import jax.numpy as jnp
from jax.experimental import pallas as pl
from jax.experimental.pallas import tpu as pltpu


def _dot_nt(p, m):
    return jax.lax.dot_general(
        p, m, (((1,), (1,)), ((), ())), preferred_element_type=jnp.float32
    )


def _m2_row(m):
    quarter = jnp.full((1, m.shape[1]), 0.25, dtype=jnp.bfloat16)
    return jax.lax.dot_general(
        quarter, m * m, (((1,), (1,)), ((), ())),
        preferred_element_type=jnp.float32,
    )


def _tile_min(dot, m2row):
    v = m2row + dot
    bk = v.shape[1]
    acc = v[:, 0:128]
    for c in range(1, bk // 128):
        acc = jnp.minimum(acc, v[:, c * 128:(c + 1) * 128])
    return acc


def _knn_body(p_ref, m_ref, out_ref, bufa, bufb, *, nq, bk):
    i = pl.program_id(0)
    p = p_ref[...]
    k = m_ref.shape[0]
    nc = k // bk
    bufs = (bufa, bufb)

    m2s = []
    acc = None

    def epilogue(c):
        nonlocal acc
        tmin = _tile_min(bufs[c % 2][...], m2s[c])
        acc = tmin if acc is None else jnp.minimum(acc, tmin)

    for c in range(nc):
        m_c = m_ref[pl.ds(c * bk, bk), :]
        bufs[c % 2][...] = _dot_nt(p, m_c)
        m2s.append(_m2_row(m_c))
        if c >= 1:
            epilogue(c - 1)
    epilogue(nc - 1)

    pf = p.astype(jnp.float32)
    p2 = jnp.sum(pf * pf, axis=1, keepdims=True)
    rowmin = jnp.min(acc, axis=1, keepdims=True)
    bmax = jnp.max(rowmin + p2)
    val = jnp.where(i == 0, bmax, jnp.maximum(out_ref[0, 0], bmax))
    out_ref[0, 0] = jnp.where(
        i == nq - 1, jnp.sqrt(jnp.maximum(val, 1e-12)), val
    )


def kernel(patches, memory_bank):
    q, d = patches.shape
    k, _ = memory_bank.shape
    bq = min(2048, q)
    bk = min(1024, k)
    nq = q // bq

    p16 = patches.astype(jnp.bfloat16)
    m16 = (memory_bank * -2.0).astype(jnp.bfloat16)

    out = pl.pallas_call(
        functools.partial(_knn_body, nq=nq, bk=bk),
        grid=(nq,),
        in_specs=[
            pl.BlockSpec((bq, d), lambda i: (i, 0)),
            pl.BlockSpec((k, d), lambda i: (0, 0)),
        ],
        out_specs=pl.BlockSpec(
            (1, 1), lambda i: (0, 0), memory_space=pltpu.SMEM
        ),
        out_shape=jax.ShapeDtypeStruct((1, 1), jnp.float32),
        scratch_shapes=[
            pltpu.VMEM((bq, bk), jnp.float32),
            pltpu.VMEM((bq, bk), jnp.float32),
        ],
        compiler_params=pltpu.CompilerParams(
            dimension_semantics=("arbitrary",),
        ),
    )(p16, m16)
    return out[0, 0]
